# baseline (speedup 1.0000x reference)
import functools
import math

import jax
import jax.numpy as jnp
from jax import lax
from jax.experimental import pallas as pl
from jax.experimental.pallas import tpu as pltpu
from jax.experimental.pallas import tpu_sc as plsc

D_MODEL = 64
SCALE = math.sqrt(D_MODEL)
CHUNK = 40


@functools.cache
def _build(R, C, V):
    info = plsc.get_sparse_core_info()
    nc, ns, lanes = info.num_cores, info.num_subcores, info.num_lanes
    nw = nc * ns
    B = R * C
    b_per_w = B // nw
    rows_per_w = R // nw
    cpr = C // CHUNK
    n_chunks = b_per_w // CHUNK
    assert n_chunks % 2 == 0 and n_chunks >= 4
    mesh = plsc.VectorSubcoreMesh(core_axis_name="c", subcore_axis_name="s")

    @functools.partial(
        pl.kernel,
        mesh=mesh,
        compiler_params=pltpu.CompilerParams(needs_layout_passes=False),
        out_type=jax.ShapeDtypeStruct((R, C, D_MODEL), jnp.float32),
        scratch_types=[
            pltpu.VMEM((b_per_w,), jnp.int32),
            pltpu.VMEM((b_per_w,), jnp.int32),
            pltpu.VMEM((CHUNK, 2 * D_MODEL), jnp.float32),
            pltpu.VMEM((CHUNK, 2 * D_MODEL), jnp.float32),
            pltpu.VMEM((CHUNK, D_MODEL), jnp.float32),
            pltpu.VMEM((CHUNK, D_MODEL), jnp.float32),
            pltpu.SemaphoreType.DMA,
            pltpu.SemaphoreType.DMA,
            pltpu.SemaphoreType.DMA,
            pltpu.SemaphoreType.DMA,
        ],
    )
    def emb_kernel(p_hbm, h_hbm, lut_hbm, out_hbm, p_v, h_v, gbuf0, gbuf1,
                   wbuf0, wbuf1, gsem0, gsem1, wsem0, wsem1):
        wid = lax.axis_index("s") * nc + lax.axis_index("c")
        row_base = wid * rows_per_w
        pltpu.sync_copy(p_hbm.at[pl.ds(wid * b_per_w, b_per_w)], p_v)
        pltpu.sync_copy(h_hbm.at[pl.ds(wid * b_per_w, b_per_w)], h_v)

        gbufs = (gbuf0, gbuf1)
        gsems = (gsem0, gsem1)
        wbufs = (wbuf0, wbuf1)
        wsems = (wsem0, wsem1)

        def gather_start(c, b):
            pltpu.async_copy(lut_hbm.at[p_v.at[pl.ds(c * CHUNK, CHUNK)]],
                             gbufs[b], gsems[b])

        def scale(c, b):
            src, dst = gbufs[b], wbufs[b]
            hbase = c * CHUNK

            @plsc.parallel_loop(0, CHUNK, unroll=4)
            def _(r):
                hh = jnp.full((lanes,), hbase + r, jnp.int32)
                hi = plsc.load_gather(h_v, [hh]) != 0
                for j in range(D_MODEL // lanes):
                    lo_sl = pl.ds(j * lanes, lanes)
                    hi_sl = pl.ds(D_MODEL + j * lanes, lanes)
                    v = jnp.where(hi, src[r, hi_sl], src[r, lo_sl])
                    dst[r, lo_sl] = v * SCALE

        def write_start(c, b):
            a = row_base + c // cpr
            m0 = (c % cpr) * CHUNK
            pltpu.async_copy(wbufs[b], out_hbm.at[a, pl.ds(m0, CHUNK)],
                             wsems[b])

        def gather_wait(b):
            pltpu.make_async_copy(lut_hbm.at[p_v.at[pl.ds(0, CHUNK)]],
                                  gbufs[b], gsems[b]).wait()

        def write_wait(b):
            pltpu.make_async_copy(wbufs[b], out_hbm.at[0, pl.ds(0, CHUNK)],
                                  wsems[b]).wait()

        gather_start(0, 0)
        gather_start(1, 1)
        gather_wait(0)
        scale(0, 0)
        write_start(0, 0)
        gather_start(2, 0)
        gather_wait(1)
        scale(1, 1)
        write_start(1, 1)

        def pair_body(t, _):
            c0 = 2 * t
            gather_start(c0 + 1, 1)
            gather_wait(0)
            write_wait(0)
            scale(c0, 0)
            write_start(c0, 0)
            gather_start(c0 + 2, 0)
            gather_wait(1)
            write_wait(1)
            scale(c0 + 1, 1)
            write_start(c0 + 1, 1)
            return ()

        lax.fori_loop(1, n_chunks // 2 - 1, pair_body, ())

        c0 = n_chunks - 2
        gather_start(c0 + 1, 1)
        gather_wait(0)
        write_wait(0)
        scale(c0, 0)
        write_start(c0, 0)
        gather_wait(1)
        write_wait(1)
        scale(c0 + 1, 1)
        write_start(c0 + 1, 1)
        write_wait(0)
        write_wait(1)

    return emb_kernel


def kernel(x, lut):
    xf = x.reshape(-1).astype(jnp.int32)
    return _build(x.shape[0], x.shape[1], lut.shape[0])(
        xf >> 1, xf & 1, lut.reshape(lut.shape[0] // 2, 2 * D_MODEL))
